# baseline (speedup 1.0000x reference)
import functools

import jax
import jax.numpy as jnp
from jax import lax
from jax.experimental import pallas as pl
from jax.experimental.pallas import tpu as pltpu
from jax.experimental.pallas import tpu_sc as plsc

NUM_CORES = 2
NUM_SUBCORES = 16
NUM_WORKERS = NUM_CORES * NUM_SUBCORES
LANES = 16
PAD_IDX = 0


def _make_lookup(n_rows, d, seq):
    assert d % LANES == 0
    assert n_rows % (NUM_WORKERS * seq) == 0
    rows_per_worker = n_rows // NUM_WORKERS
    chunks = rows_per_worker // seq
    d_slices = d // LANES

    mesh = plsc.VectorSubcoreMesh(core_axis_name="c", subcore_axis_name="s")

    @functools.partial(
        pl.kernel,
        mesh=mesh,
        compiler_params=pltpu.CompilerParams(use_tc_tiling_on_sc=False),
        out_type=jax.ShapeDtypeStruct((n_rows, d), jnp.float32),
        scratch_types=[
            pltpu.VMEM((seq,), jnp.int32),
            pltpu.VMEM((seq, d), jnp.float32),
            pltpu.VMEM((seq, d), jnp.float32),
            pltpu.SemaphoreType.DMA,
        ],
    )
    def body(x_hbm, table_hbm, pe_hbm, out_hbm, idx_v, rows_v, pe_v, sem):
        cid = lax.axis_index("c")
        sid = lax.axis_index("s")
        wid = sid * NUM_CORES + cid
        base = wid * rows_per_worker

        pltpu.sync_copy(pe_hbm, pe_v)

        def chunk_body(g, carry):
            rowbase = base + g * seq
            pltpu.sync_copy(x_hbm.at[pl.ds(rowbase, seq)], idx_v)
            cp1 = pltpu.make_async_copy(
                table_hbm.at[idx_v.at[pl.ds(0, 128)]],
                rows_v.at[pl.ds(0, 128)],
                sem,
            )
            cp1.start()
            cp2 = pltpu.make_async_copy(
                table_hbm.at[idx_v.at[pl.ds(128, seq - 128)]],
                rows_v.at[pl.ds(128, seq - 128)],
                sem,
            )
            cp2.start()
            cp1.wait()
            cp2.wait()

            def add_row(r, c):
                for j in range(d_slices):
                    sl = pl.ds(j * LANES, LANES)
                    rows_v[r, sl] = rows_v[r, sl] + pe_v[r, sl]
                return c

            lax.fori_loop(0, seq, add_row, 0)

            zmask = idx_v[pl.ds(0, LANES)] == PAD_IDX
            for gi in range(1, seq // LANES):
                zmask = zmask | (idx_v[pl.ds(gi * LANES, LANES)] == PAD_IDX)
            if seq % LANES:
                zmask = zmask | (idx_v[pl.ds(seq - LANES, LANES)] == PAD_IDX)
            zi = jnp.where(zmask, 1, 0)
            has_pad = zi[0]
            for li in range(1, LANES):
                has_pad = has_pad | zi[li]

            @pl.when(has_pad != 0)
            def _fix():
                n_groups = -(-seq // LANES)

                def fix_group(gi, c):
                    off = jnp.minimum(gi * LANES, seq - LANES)
                    v = idx_v[pl.ds(off, LANES)]
                    for li in range(LANES):
                        s = v[li]

                        @pl.when(s == PAD_IDX)
                        def _zero_row():
                            r = off + li
                            for j in range(d_slices):
                                sl = pl.ds(j * LANES, LANES)
                                rows_v[r, sl] = pe_v[r, sl]

                    return c

                lax.fori_loop(0, n_groups, fix_group, 0)

            pltpu.sync_copy(rows_v, out_hbm.at[pl.ds(rowbase, seq)])
            return carry

        lax.fori_loop(0, chunks, chunk_body, 0)

    return body


def kernel(x, table, pe):
    b, l = x.shape
    d = table.shape[1]
    xf = x.reshape(b * l)
    pe_l = pe[:l]
    lookup = _make_lookup(b * l, d, l)
    out = lookup(xf, table, pe_l)
    return out.reshape(b, l, d)


# speedup vs baseline: 1.2122x; 1.2122x over previous
import functools

import jax
import jax.numpy as jnp
from jax import lax
from jax.experimental import pallas as pl
from jax.experimental.pallas import tpu as pltpu
from jax.experimental.pallas import tpu_sc as plsc

NUM_CORES = 2
NUM_SUBCORES = 16
NUM_WORKERS = NUM_CORES * NUM_SUBCORES
LANES = 16
PAD_IDX = 0
NBUF = 4
LOOKAHEAD = 2


def _make_lookup(n_rows, d, seq):
    assert d % LANES == 0
    assert n_rows % (NUM_WORKERS * seq) == 0
    rows_per_worker = n_rows // NUM_WORKERS
    chunks = rows_per_worker // seq
    assert chunks % NBUF == 0
    d_slices = d // LANES
    gather_splits = [(0, 128), (128, seq - 128)] if seq > 128 else [(0, seq)]

    mesh = plsc.VectorSubcoreMesh(core_axis_name="c", subcore_axis_name="s")

    @functools.partial(
        pl.kernel,
        mesh=mesh,
        compiler_params=pltpu.CompilerParams(use_tc_tiling_on_sc=False),
        out_type=jax.ShapeDtypeStruct((n_rows, d), jnp.float32),
        scratch_types=[
            pltpu.VMEM((rows_per_worker,), jnp.int32),
            pltpu.VMEM((NBUF, seq, d), jnp.float32),
            pltpu.VMEM((seq, d), jnp.float32),
            pltpu.SemaphoreType.DMA((NBUF,)),
            pltpu.SemaphoreType.DMA((NBUF,)),
        ],
    )
    def body(x_hbm, table_hbm, pe_hbm, out_hbm, idx_all, rows_v, pe_v, gsem, ssem):
        cid = lax.axis_index("c")
        sid = lax.axis_index("s")
        wid = sid * NUM_CORES + cid
        base = wid * rows_per_worker

        pltpu.sync_copy(pe_hbm, pe_v)
        pltpu.sync_copy(x_hbm.at[pl.ds(base, rows_per_worker)], idx_all)

        def gather_copies(g, b):
            off = g * seq
            return [
                pltpu.make_async_copy(
                    table_hbm.at[idx_all.at[pl.ds(off + s0, sn)]],
                    rows_v.at[b].at[pl.ds(s0, sn)],
                    gsem.at[b],
                )
                for s0, sn in gather_splits
            ]

        def store_copy(g, b):
            return pltpu.make_async_copy(
                rows_v.at[b], out_hbm.at[pl.ds(base + g * seq, seq)], ssem.at[b]
            )

        def start_gather(g, b):
            for cp in gather_copies(g, b):
                cp.start()

        def compute(g, b):
            @plsc.parallel_loop(0, seq, unroll=2)
            def _add_row(r):
                for j in range(d_slices):
                    sl = pl.ds(j * LANES, LANES)
                    rows_v[b, r, sl] = rows_v[b, r, sl] + pe_v[r, sl]

            off = g * seq
            zmask = idx_all[pl.ds(off, LANES)] == PAD_IDX
            for gi in range(1, seq // LANES):
                zmask = zmask | (idx_all[pl.ds(off + gi * LANES, LANES)] == PAD_IDX)
            if seq % LANES:
                zmask = zmask | (idx_all[pl.ds(off + seq - LANES, LANES)] == PAD_IDX)
            zi = jnp.where(zmask, 1, 0)
            has_pad = zi[0]
            for li in range(1, LANES):
                has_pad = has_pad | zi[li]

            @pl.when(has_pad != 0)
            def _fix():
                n_groups = -(-seq // LANES)

                def fix_group(gi, c):
                    goff = jnp.minimum(gi * LANES, seq - LANES)
                    v = idx_all[pl.ds(off + goff, LANES)]
                    for li in range(LANES):
                        s = v[li]

                        @pl.when(s == PAD_IDX)
                        def _zero_row():
                            r = goff + li
                            for j in range(d_slices):
                                sl = pl.ds(j * LANES, LANES)
                                rows_v[b, r, sl] = pe_v[r, sl]

                    return c

                lax.fori_loop(0, n_groups, fix_group, 0)

        for g in range(LOOKAHEAD):
            start_gather(g, g % NBUF)

        def quad(t, carry):
            for bb in range(NBUF):
                g = t * NBUF + bb
                b = (g + LOOKAHEAD) % NBUF

                @pl.when(g + LOOKAHEAD < chunks)
                def _refill():
                    @pl.when(g + LOOKAHEAD >= NBUF)
                    def _drain():
                        store_copy(g + LOOKAHEAD - NBUF, b).wait()

                    start_gather(g + LOOKAHEAD, b)

                for cp in gather_copies(g, bb):
                    cp.wait()
                compute(g, bb)
                store_copy(g, bb).start()
            return carry

        lax.fori_loop(0, chunks // NBUF, quad, 0)

        for g in range(chunks - NBUF, chunks):
            store_copy(g, g % NBUF).wait()

    return body


def kernel(x, table, pe):
    b, l = x.shape
    d = table.shape[1]
    xf = x.reshape(b * l)
    pe_l = pe[:l]
    lookup = _make_lookup(b * l, d, l)
    out = lookup(xf, table, pe_l)
    return out.reshape(b, l, d)
